# Initial kernel scaffold; baseline (speedup 1.0000x reference)
import functools

import jax
import jax.numpy as jnp
from jax import lax
from jax.experimental import pallas as pl
from jax.experimental.pallas import tpu as pltpu
from jax.experimental.pallas import tpu_sc as plsc

NBINS = 256
MAX_VAL = 640.0
PIXELS = 4096.0
LUT_SCALE = 255.0 / 4096.0
NW = 32
BANDS = 384
BPW = BANDS // NW
BAND_ELEMS = 64 * 512
LROW = 8 * NBINS

_mesh = plsc.VectorSubcoreMesh(core_axis_name="c", subcore_axis_name="s")


@functools.partial(
    pl.kernel,
    out_type=jax.ShapeDtypeStruct((BANDS * LROW,), jnp.float32),
    mesh=_mesh,
    scratch_types=[
        pltpu.VMEM((BAND_ELEMS,), jnp.float32),
        pltpu.VMEM((16 * LROW,), jnp.float32),
        pltpu.VMEM((LROW,), jnp.float32),
        pltpu.VMEM((16,), jnp.float32),
    ],
)
def _hist_lut_kernel(img_hbm, luts_hbm, band_v, hist_v, lut_v, tmp_v):
    wid = lax.axis_index("s") * 2 + lax.axis_index("c")
    lanes = lax.iota(jnp.int32, 16)
    lane_off = lanes * LROW
    ones = jnp.ones((16,), jnp.float32)
    zeros = jnp.zeros((16,), jnp.float32)
    last = jnp.full((16,), 15, jnp.int32)

    def band_body(t, _):
        band = wid * BPW + t
        pltpu.sync_copy(img_hbm.at[pl.ds(band * BAND_ELEMS, BAND_ELEMS)], band_v)

        def zero_body(i, _):
            hist_v[pl.ds(i * 16, 16)] = zeros
            return 0
        lax.fori_loop(0, 16 * LROW // 16, zero_body, 0)

        def scat_body(i, _):
            pix = band_v[pl.ds(i * 16, 16)]
            b = (pix * 256.0).astype(jnp.int32)
            b = jnp.minimum(jnp.maximum(b, 0), 255)
            tile = (i % 32) // 4
            idx = b + tile * NBINS + lane_off
            plsc.addupdate_scatter(hist_v, [idx], ones)
            return 0
        lax.fori_loop(0, BAND_ELEMS // 16, scat_body, 0)

        def tile_body(k, _):
            def merge_body(c, acc):
                base = k * NBINS + c * 16
                s = hist_v[pl.ds(base, 16)]
                for l in range(1, 16):
                    s = s + hist_v[pl.ds(l * LROW + base, 16)]
                s = jnp.minimum(s, MAX_VAL)
                lut_v[pl.ds(base, 16)] = s
                return acc + s
            acc = lax.fori_loop(0, 16, merge_body, zeros)
            tmp_v[...] = plsc.cumsum(acc)
            total = plsc.load_gather(tmp_v, [last])
            clipped = PIXELS - total
            redist = jnp.floor(clipped * (1.0 / 256.0))
            residual = clipped - redist * 256.0

            def lut_body(c, carry):
                base = k * NBINS + c * 16
                v = lut_v[pl.ds(base, 16)]
                binid = (c * 16 + lanes).astype(jnp.float32)
                v = v + redist + jnp.where(binid < residual, 1.0, 0.0)
                cumv = plsc.cumsum(v) + carry
                tmp_v[...] = cumv
                carry = plsc.load_gather(tmp_v, [last])
                lut = jnp.floor(jnp.clip(cumv * LUT_SCALE, 0.0, 255.0))
                lut_v[pl.ds(base, 16)] = lut * (1.0 / 255.0)
                return carry
            lax.fori_loop(0, 16, lut_body, zeros)
            return 0
        lax.fori_loop(0, 8, tile_body, 0)
        pltpu.sync_copy(lut_v, luts_hbm.at[pl.ds(band * LROW, LROW)])
        return 0
    lax.fori_loop(0, BPW, band_body, 0)


@functools.partial(
    pl.kernel,
    out_type=jax.ShapeDtypeStruct((BANDS * BAND_ELEMS,), jnp.float32),
    mesh=_mesh,
    scratch_types=[
        pltpu.VMEM((BAND_ELEMS,), jnp.float32),
        pltpu.VMEM((BAND_ELEMS,), jnp.float32),
        pltpu.VMEM((3 * LROW,), jnp.float32),
        pltpu.VMEM((64,), jnp.float32),
    ],
)
def _apply_kernel(img_hbm, luts_hbm, out_hbm, band_v, out_v, lutv, wx_v):
    wid = lax.axis_index("s") * 2 + lax.axis_index("c")
    lanes = lax.iota(jnp.int32, 16)
    for q in range(4):
        m = (q * 16 + lanes).astype(jnp.float32)
        wx_v[pl.ds(q * 16, 16)] = (m + 0.5) * (1.0 / 64.0) + (0.5 if q < 2 else -0.5)

    def band_body(t, _):
        band = wid * BPW + t
        p = band // 8
        j = band % 8
        pltpu.sync_copy(img_hbm.at[pl.ds(band * BAND_ELEMS, BAND_ELEMS)], band_v)
        jm = jnp.maximum(j - 1, 0)
        jp = jnp.minimum(j + 1, 7)
        pltpu.sync_copy(luts_hbm.at[pl.ds((p * 8 + jm) * LROW, LROW)],
                        lutv.at[pl.ds(0, LROW)])
        pltpu.sync_copy(luts_hbm.at[pl.ds(band * LROW, LROW)],
                        lutv.at[pl.ds(LROW, LROW)])
        pltpu.sync_copy(luts_hbm.at[pl.ds((p * 8 + jp) * LROW, LROW)],
                        lutv.at[pl.ds(2 * LROW, LROW)])

        def row_body(r, _):
            rv = jnp.full((16,), r, jnp.int32)
            ltv = rv < 32
            wy = (rv.astype(jnp.float32) * (1.0 / 64.0) + 0.0078125
                  + jnp.where(ltv, 0.5, -0.5))
            ybase = jnp.where(r < 32, 0, LROW)

            def g_body(g, _):
                kk = g // 4
                q = g % 4
                x0 = kk - 1 + q // 2
                x0c = jnp.maximum(x0, 0)
                x1c = jnp.minimum(x0 + 1, 7)
                wx = wx_v[pl.ds(q * 16, 16)]
                pix = band_v[pl.ds(r * 512 + g * 16, 16)]
                b = (pix * 256.0).astype(jnp.int32)
                b = jnp.minimum(jnp.maximum(b, 0), 255)
                b00 = b + (ybase + x0c * NBINS)
                b01 = b + (ybase + x1c * NBINS)
                b10 = b + (ybase + LROW + x0c * NBINS)
                b11 = b + (ybase + LROW + x1c * NBINS)
                g00 = plsc.load_gather(lutv, [b00])
                g01 = plsc.load_gather(lutv, [b01])
                g10 = plsc.load_gather(lutv, [b10])
                g11 = plsc.load_gather(lutv, [b11])
                top = g00 + wx * (g01 - g00)
                bot = g10 + wx * (g11 - g10)
                out_v[pl.ds(r * 512 + g * 16, 16)] = top + wy * (bot - top)
                return 0
            lax.fori_loop(0, 32, g_body, 0)
            return 0
        lax.fori_loop(0, 64, row_body, 0)
        pltpu.sync_copy(out_v, out_hbm.at[pl.ds(band * BAND_ELEMS, BAND_ELEMS)])
        return 0
    lax.fori_loop(0, BPW, band_body, 0)


def kernel(img):
    B, C, H, W = img.shape
    flat = img.reshape(BANDS * BAND_ELEMS)
    luts = _hist_lut_kernel(flat)
    out = _apply_kernel(flat, luts)
    return out.reshape(B, C, H, W)


# speedup vs baseline: 658.7415x; 658.7415x over previous
import functools

import jax
import jax.numpy as jnp
from jax import lax
from jax.experimental import pallas as pl
from jax.experimental.pallas import tpu as pltpu
from jax.experimental.pallas import tpu_sc as plsc

NBINS = 256
MAX_VAL = 640.0
PIXELS = 4096.0
LUT_SCALE = 255.0 / 4096.0
NW = 32
BANDS = 384
BPW = BANDS // NW
BAND_ELEMS = 64 * 512
LROW = 8 * NBINS
HCOPIES = 16

_mesh = plsc.VectorSubcoreMesh(core_axis_name="c", subcore_axis_name="s")


@functools.partial(
    pl.kernel,
    out_type=jax.ShapeDtypeStruct((BANDS * LROW,), jnp.float32),
    mesh=_mesh,
    compiler_params=pltpu.CompilerParams(needs_layout_passes=False),
    scratch_types=[
        pltpu.VMEM((BAND_ELEMS,), jnp.float32),
        pltpu.VMEM((HCOPIES * LROW,), jnp.float32),
        pltpu.VMEM((LROW,), jnp.float32),
        pltpu.VMEM((16,), jnp.float32),
    ],
)
def _hist_lut_kernel(img_hbm, luts_hbm, band_v, hist_v, lut_v, tmp_v):
    wid = lax.axis_index("s") * 2 + lax.axis_index("c")
    lanes = lax.iota(jnp.int32, 16)
    lane_off = lanes * LROW
    tile_base = [lane_off + k * NBINS for k in range(8)]
    ones = jnp.ones((16,), jnp.float32)
    zeros = jnp.zeros((16,), jnp.float32)
    last = jnp.full((16,), 15, jnp.int32)

    def zero_body(i, _):
        hist_v[pl.ds(i * 16, 16)] = zeros
        return 0
    lax.fori_loop(0, HCOPIES * LROW // 16, zero_body, 0)

    def band_body(t, _):
        band = wid * BPW + t
        pltpu.sync_copy(img_hbm.at[pl.ds(band * BAND_ELEMS, BAND_ELEMS)], band_v)

        def row_body(r, _):
            rbase = r * 512
            for g in range(32):
                pix = band_v[pl.ds(rbase + g * 16, 16)]
                b = (pix * 256.0).astype(jnp.int32)
                b = jnp.minimum(b, 255)
                plsc.addupdate_scatter(hist_v, [b + tile_base[g // 4]], ones)
            return 0
        lax.fori_loop(0, 64, row_body, 0)

        for k in range(8):
            def merge_body(c, acc):
                base = k * NBINS + c * 16
                s = hist_v[pl.ds(base, 16)]
                hist_v[pl.ds(base, 16)] = zeros
                for l in range(1, HCOPIES):
                    s = s + hist_v[pl.ds(l * LROW + base, 16)]
                    hist_v[pl.ds(l * LROW + base, 16)] = zeros
                s = jnp.minimum(s, MAX_VAL)
                lut_v[pl.ds(base, 16)] = s
                return acc + s
            acc = lax.fori_loop(0, 16, merge_body, zeros)
            tmp_v[...] = plsc.cumsum(acc)
            total = plsc.load_gather(tmp_v, [last])
            clipped = PIXELS - total
            redist = (clipped * (1.0 / 256.0)).astype(jnp.int32).astype(jnp.float32)
            residual = clipped - redist * 256.0

            def lut_body(c, carry):
                base = k * NBINS + c * 16
                v = lut_v[pl.ds(base, 16)]
                binid = (c * 16 + lanes).astype(jnp.float32)
                v = v + redist + jnp.where(binid < residual, 1.0, 0.0)
                cumv = plsc.cumsum(v) + carry
                tmp_v[...] = cumv
                carry = plsc.load_gather(tmp_v, [last])
                lut = (jnp.clip(cumv * LUT_SCALE, 0.0, 255.0)
                       ).astype(jnp.int32).astype(jnp.float32)
                lut_v[pl.ds(base, 16)] = lut
                return carry
            lax.fori_loop(0, 16, lut_body, zeros)
        pltpu.sync_copy(lut_v, luts_hbm.at[pl.ds(band * LROW, LROW)])
        return 0
    lax.fori_loop(0, BPW, band_body, 0)


@functools.partial(
    pl.kernel,
    out_type=jax.ShapeDtypeStruct((BANDS * BAND_ELEMS,), jnp.float32),
    mesh=_mesh,
    compiler_params=pltpu.CompilerParams(needs_layout_passes=False),
    scratch_types=[
        pltpu.VMEM((BAND_ELEMS,), jnp.float32),
        pltpu.VMEM((BAND_ELEMS,), jnp.float32),
        pltpu.VMEM((3 * LROW,), jnp.float32),
        pltpu.VMEM((2 * LROW,), jnp.int32),
    ],
)
def _apply_kernel(img_hbm, luts_hbm, out_hbm, band_v, out_v, lutrows_v, pck_v):
    wid = lax.axis_index("s") * 2 + lax.axis_index("c")
    lanes = lax.iota(jnp.int32, 16)
    lanes_f = lanes.astype(jnp.float32)
    wxs = [(q * 16 + lanes_f + 0.5) * (1.0 / 64.0) + (0.5 if q < 2 else -0.5)
           for q in range(4)]
    himask = jnp.full((16,), -65536, jnp.int32)

    def band_body(t, _):
        band = wid * BPW + t
        p = band // 8
        j = band % 8
        pltpu.sync_copy(img_hbm.at[pl.ds(band * BAND_ELEMS, BAND_ELEMS)], band_v)
        jm = jnp.maximum(j - 1, 0)
        jp = jnp.minimum(j + 1, 7)
        pltpu.sync_copy(luts_hbm.at[pl.ds((p * 8 + jm) * LROW, LROW)],
                        lutrows_v.at[pl.ds(0, LROW)])
        pltpu.sync_copy(luts_hbm.at[pl.ds(band * LROW, LROW)],
                        lutrows_v.at[pl.ds(LROW, LROW)])
        pltpu.sync_copy(luts_hbm.at[pl.ds((p * 8 + jp) * LROW, LROW)],
                        lutrows_v.at[pl.ds(2 * LROW, LROW)])

        def pack_body(i, _):
            off = i * 16
            a = plsc.bitcast(lutrows_v[pl.ds(off, 16)], jnp.int32)
            b2 = plsc.bitcast(lutrows_v[pl.ds(LROW + off, 16)], jnp.int32)
            c2 = plsc.bitcast(lutrows_v[pl.ds(2 * LROW + off, 16)], jnp.int32)
            bhi = jax.lax.shift_right_logical(b2, 16)
            chi = jax.lax.shift_right_logical(c2, 16)
            pck_v[pl.ds(off, 16)] = (a & himask) | bhi
            pck_v[pl.ds(LROW + off, 16)] = (b2 & himask) | chi
            return 0
        lax.fori_loop(0, LROW // 16, pack_body, 0)

        def row_body(r, _):
            rbase = r * 512
            rv = jnp.full((16,), r, jnp.int32)
            ltv = rv < 32
            wy = (rv.astype(jnp.float32) * (1.0 / 64.0) + 0.0078125
                  + jnp.where(ltv, 0.5, -0.5))
            ybase = jnp.where(r < 32, 0, LROW)
            combo = [jnp.full((16,), ybase + x * NBINS, jnp.int32)
                     for x in range(8)]
            for g in range(32):
                k = g // 4
                q = g % 4
                x0 = k - 1 + q // 2
                x0c = max(x0, 0)
                x1c = min(x0 + 1, 7)
                pix = band_v[pl.ds(rbase + g * 16, 16)]
                b = (pix * 256.0).astype(jnp.int32)
                b = jnp.minimum(b, 255)
                w0 = plsc.load_gather(pck_v, [b + combo[x0c]])
                w1 = plsc.load_gather(pck_v, [b + combo[x1c]])
                y00 = plsc.bitcast(w0 & himask, jnp.float32)
                y10 = plsc.bitcast(jax.lax.shift_left(w0, 16), jnp.float32)
                y01 = plsc.bitcast(w1 & himask, jnp.float32)
                y11 = plsc.bitcast(jax.lax.shift_left(w1, 16), jnp.float32)
                wx = wxs[q]
                top = y00 + wx * (y01 - y00)
                bot = y10 + wx * (y11 - y10)
                out_v[pl.ds(rbase + g * 16, 16)] = (
                    (top + wy * (bot - top)) * (1.0 / 255.0))
            return 0
        lax.fori_loop(0, 64, row_body, 0)
        pltpu.sync_copy(out_v, out_hbm.at[pl.ds(band * BAND_ELEMS, BAND_ELEMS)])
        return 0
    lax.fori_loop(0, BPW, band_body, 0)


def kernel(img):
    B, C, H, W = img.shape
    flat = img.reshape(BANDS * BAND_ELEMS)
    luts = _hist_lut_kernel(flat)
    out = _apply_kernel(flat, luts)
    return out.reshape(B, C, H, W)


# speedup vs baseline: 1557.6394x; 2.3646x over previous
import functools

import jax
import jax.numpy as jnp
from jax import lax
from jax.experimental import pallas as pl
from jax.experimental.pallas import tpu as pltpu
from jax.experimental.pallas import tpu_sc as plsc

NBINS = 256
MAX_VAL = 640.0
PIXELS = 4096.0
LUT_SCALE = 255.0 / 4096.0
NW = 32
BANDS = 384
BPW = BANDS // NW
BAND_ELEMS = 64 * 512
LROW = 8 * NBINS
HCOPIES = 16
GROUP = 8

_mesh = plsc.VectorSubcoreMesh(core_axis_name="c", subcore_axis_name="s")


_GATHER_DNUMS = lax.GatherDimensionNumbers(
    offset_dims=(), collapsed_slice_dims=(0,), start_index_map=(0,))


def _take_last(v):
    last = jnp.full((16, 1), 15, jnp.int32)
    return lax.gather(v, last, _GATHER_DNUMS, slice_sizes=(1,),
                      mode=lax.GatherScatterMode.PROMISE_IN_BOUNDS)


@functools.partial(
    pl.kernel,
    out_type=jax.ShapeDtypeStruct((BANDS * LROW,), jnp.float32),
    mesh=_mesh,
    compiler_params=pltpu.CompilerParams(needs_layout_passes=False),
    scratch_types=[
        pltpu.VMEM((BAND_ELEMS,), jnp.float32),
        pltpu.VMEM((HCOPIES * LROW,), jnp.float32),
        pltpu.VMEM((LROW,), jnp.float32),
    ],
)
def _hist_lut_kernel(img_hbm, luts_hbm, band_v, hist_v, lut_v):
    wid = lax.axis_index("s") * 2 + lax.axis_index("c")
    lanes = lax.iota(jnp.int32, 16)
    lane_off = lanes * LROW
    tile_base = [lane_off + k * NBINS for k in range(8)]
    ones = jnp.ones((16,), jnp.float32)
    zeros = jnp.zeros((16,), jnp.float32)

    def zero_body(i, _):
        hist_v[pl.ds(i * 16, 16)] = zeros
        return 0
    lax.fori_loop(0, HCOPIES * LROW // 16, zero_body, 0)

    def band_body(t, _):
        band = wid * BPW + t
        pltpu.sync_copy(img_hbm.at[pl.ds(band * BAND_ELEMS, BAND_ELEMS)], band_v)

        def row_body(r, _):
            rbase = r * 512
            for g0 in range(0, 32, GROUP):
                gs = range(g0, g0 + GROUP)
                pixs = [band_v[pl.ds(rbase + g * 16, 16)] for g in gs]
                idxs = [jnp.minimum((px * 256.0).astype(jnp.int32), 255)
                        + tile_base[g // 4] for g, px in zip(gs, pixs)]
                for idx in idxs:
                    plsc.addupdate_scatter(hist_v, [idx], ones)
            return 0
        lax.fori_loop(0, 64, row_body, 0)

        def tile_body(k, _):
            def merge_body(c, acc):
                base = k * NBINS + c * 16
                vs = [hist_v[pl.ds(l * LROW + base, 16)] for l in range(HCOPIES)]
                for l in range(HCOPIES):
                    hist_v[pl.ds(l * LROW + base, 16)] = zeros
                while len(vs) > 1:
                    vs = [a + b for a, b in zip(vs[0::2], vs[1::2])]
                s = jnp.minimum(vs[0], MAX_VAL)
                lut_v[pl.ds(base, 16)] = s
                return acc + s
            acc = lax.fori_loop(0, 16, merge_body, zeros)
            total = _take_last(plsc.cumsum(acc))
            clipped = PIXELS - total
            redist = (clipped * (1.0 / 256.0)).astype(jnp.int32).astype(jnp.float32)
            residual = clipped - redist * 256.0

            def lut_body(c, carry):
                base = k * NBINS + c * 16
                v = lut_v[pl.ds(base, 16)]
                binid = (c * 16 + lanes).astype(jnp.float32)
                v = v + redist + jnp.where(binid < residual, 1.0, 0.0)
                cumv = plsc.cumsum(v) + carry
                carry = _take_last(cumv)
                lut = (jnp.clip(cumv * LUT_SCALE, 0.0, 255.0)
                       ).astype(jnp.int32).astype(jnp.float32)
                lut_v[pl.ds(base, 16)] = lut
                return carry
            lax.fori_loop(0, 16, lut_body, zeros)
            return 0
        lax.fori_loop(0, 8, tile_body, 0)
        pltpu.sync_copy(lut_v, luts_hbm.at[pl.ds(band * LROW, LROW)])
        return 0
    lax.fori_loop(0, BPW, band_body, 0)


@functools.partial(
    pl.kernel,
    out_type=jax.ShapeDtypeStruct((BANDS * BAND_ELEMS,), jnp.float32),
    mesh=_mesh,
    compiler_params=pltpu.CompilerParams(needs_layout_passes=False),
    scratch_types=[
        pltpu.VMEM((BAND_ELEMS,), jnp.float32),
        pltpu.VMEM((BAND_ELEMS,), jnp.float32),
        pltpu.VMEM((3 * LROW,), jnp.float32),
        pltpu.VMEM((2 * LROW,), jnp.int32),
    ],
)
def _apply_kernel(img_hbm, luts_hbm, out_hbm, band_v, out_v, lutrows_v, pck_v):
    wid = lax.axis_index("s") * 2 + lax.axis_index("c")
    lanes = lax.iota(jnp.int32, 16)
    lanes_f = lanes.astype(jnp.float32)
    wxs = [(q * 16 + lanes_f + 0.5) * (1.0 / 64.0) + (0.5 if q < 2 else -0.5)
           for q in range(4)]
    himask = jnp.full((16,), -65536, jnp.int32)

    def band_body(t, _):
        band = wid * BPW + t
        p = band // 8
        j = band % 8
        pltpu.sync_copy(img_hbm.at[pl.ds(band * BAND_ELEMS, BAND_ELEMS)], band_v)
        jm = jnp.maximum(j - 1, 0)
        jp = jnp.minimum(j + 1, 7)
        pltpu.sync_copy(luts_hbm.at[pl.ds((p * 8 + jm) * LROW, LROW)],
                        lutrows_v.at[pl.ds(0, LROW)])
        pltpu.sync_copy(luts_hbm.at[pl.ds(band * LROW, LROW)],
                        lutrows_v.at[pl.ds(LROW, LROW)])
        pltpu.sync_copy(luts_hbm.at[pl.ds((p * 8 + jp) * LROW, LROW)],
                        lutrows_v.at[pl.ds(2 * LROW, LROW)])

        def pack_body(i, _):
            off = i * 32
            a0 = plsc.bitcast(lutrows_v[pl.ds(off, 16)], jnp.int32)
            a1 = plsc.bitcast(lutrows_v[pl.ds(off + 16, 16)], jnp.int32)
            b0 = plsc.bitcast(lutrows_v[pl.ds(LROW + off, 16)], jnp.int32)
            b1 = plsc.bitcast(lutrows_v[pl.ds(LROW + off + 16, 16)], jnp.int32)
            c0 = plsc.bitcast(lutrows_v[pl.ds(2 * LROW + off, 16)], jnp.int32)
            c1 = plsc.bitcast(lutrows_v[pl.ds(2 * LROW + off + 16, 16)], jnp.int32)
            pck_v[pl.ds(off, 16)] = (a0 & himask) | jax.lax.shift_right_logical(b0, 16)
            pck_v[pl.ds(off + 16, 16)] = (a1 & himask) | jax.lax.shift_right_logical(b1, 16)
            pck_v[pl.ds(LROW + off, 16)] = (b0 & himask) | jax.lax.shift_right_logical(c0, 16)
            pck_v[pl.ds(LROW + off + 16, 16)] = (b1 & himask) | jax.lax.shift_right_logical(c1, 16)
            return 0
        lax.fori_loop(0, LROW // 32, pack_body, 0)

        def row_body(r, _):
            rbase = r * 512
            rv = jnp.full((16,), r, jnp.int32)
            ltv = rv < 32
            wy = (rv.astype(jnp.float32) * (1.0 / 64.0) + 0.0078125
                  + jnp.where(ltv, 0.5, -0.5))
            ybase = jnp.where(r < 32, 0, LROW)
            combo = [jnp.full((16,), ybase + x * NBINS, jnp.int32)
                     for x in range(8)]
            for g0 in range(0, 32, GROUP):
                gs = list(range(g0, g0 + GROUP))
                pixs = [band_v[pl.ds(rbase + g * 16, 16)] for g in gs]
                bins = [jnp.minimum((px * 256.0).astype(jnp.int32), 255)
                        for px in pixs]
                w0s, w1s = [], []
                for g, b in zip(gs, bins):
                    k = g // 4
                    q = g % 4
                    x0 = k - 1 + q // 2
                    x0c = max(x0, 0)
                    x1c = min(x0 + 1, 7)
                    w0s.append(plsc.load_gather(pck_v, [b + combo[x0c]]))
                    w1s.append(plsc.load_gather(pck_v, [b + combo[x1c]]))
                for g, w0, w1 in zip(gs, w0s, w1s):
                    y00 = plsc.bitcast(w0 & himask, jnp.float32)
                    y10 = plsc.bitcast(jax.lax.shift_left(w0, 16), jnp.float32)
                    y01 = plsc.bitcast(w1 & himask, jnp.float32)
                    y11 = plsc.bitcast(jax.lax.shift_left(w1, 16), jnp.float32)
                    wx = wxs[g % 4]
                    top = y00 + wx * (y01 - y00)
                    bot = y10 + wx * (y11 - y10)
                    out_v[pl.ds(rbase + g * 16, 16)] = (
                        (top + wy * (bot - top)) * (1.0 / 255.0))
            return 0
        lax.fori_loop(0, 64, row_body, 0)
        pltpu.sync_copy(out_v, out_hbm.at[pl.ds(band * BAND_ELEMS, BAND_ELEMS)])
        return 0
    lax.fori_loop(0, BPW, band_body, 0)


def kernel(img):
    B, C, H, W = img.shape
    flat = img.reshape(BANDS * BAND_ELEMS)
    luts = _hist_lut_kernel(flat)
    out = _apply_kernel(flat, luts)
    return out.reshape(B, C, H, W)


# speedup vs baseline: 1833.5980x; 1.1772x over previous
import functools

import jax
import jax.numpy as jnp
from jax import lax
from jax.experimental import pallas as pl
from jax.experimental.pallas import tpu as pltpu
from jax.experimental.pallas import tpu_sc as plsc

NBINS = 256
MAX_VAL = 640.0
PIXELS = 4096.0
LUT_SCALE = 255.0 / 4096.0
NW = 32
BANDS = 384
BPW = BANDS // NW
BAND_ELEMS = 64 * 512
LROW = 8 * NBINS
HCOPIES = 16
GROUP = 8

_mesh = plsc.VectorSubcoreMesh(core_axis_name="c", subcore_axis_name="s")

_GATHER_DNUMS = lax.GatherDimensionNumbers(
    offset_dims=(), collapsed_slice_dims=(0,), start_index_map=(0,))


def _take_last(v):
    last = jnp.full((16, 1), 15, jnp.int32)
    return lax.gather(v, last, _GATHER_DNUMS, slice_sizes=(1,),
                      mode=lax.GatherScatterMode.PROMISE_IN_BOUNDS)


@functools.partial(
    pl.kernel,
    out_type=jax.ShapeDtypeStruct((BANDS * LROW,), jnp.float32),
    mesh=_mesh,
    compiler_params=pltpu.CompilerParams(needs_layout_passes=False),
    scratch_types=[
        pltpu.VMEM((BAND_ELEMS,), jnp.float32),
        pltpu.VMEM((BAND_ELEMS,), jnp.float32),
        pltpu.VMEM((HCOPIES * LROW,), jnp.float32),
        pltpu.VMEM((LROW,), jnp.float32),
        pltpu.SemaphoreType.DMA,
        pltpu.SemaphoreType.DMA,
        pltpu.SemaphoreType.DMA,
    ],
)
def _hist_lut_kernel(img_hbm, luts_hbm, band_a, band_b, hist_v, lut_v,
                     sem_a, sem_b, sem_o):
    wid = lax.axis_index("s") * 2 + lax.axis_index("c")
    lanes = lax.iota(jnp.int32, 16)
    lane_off = lanes * LROW
    tile_base = [lane_off + k * NBINS for k in range(8)]
    ones = jnp.ones((16,), jnp.float32)
    zeros = jnp.zeros((16,), jnp.float32)

    def in_slice(t):
        band = wid * BPW + t
        return img_hbm.at[pl.ds(band * BAND_ELEMS, BAND_ELEMS)]

    pltpu.async_copy(in_slice(0), band_a, sem_a)
    pltpu.async_copy(in_slice(1), band_b, sem_b)

    def zero_body(i, _):
        hist_v[pl.ds(i * 16, 16)] = zeros
        return 0
    lax.fori_loop(0, HCOPIES * LROW // 16, zero_body, 0)

    def half(t, buf, sem):
        band = wid * BPW + t
        pltpu.make_async_copy(in_slice(t), buf, sem).wait()

        def row_body(r, _):
            rbase = r * 512
            for g0 in range(0, 32, GROUP):
                gs = range(g0, g0 + GROUP)
                pixs = [buf[pl.ds(rbase + g * 16, 16)] for g in gs]
                idxs = [jnp.minimum((px * 256.0).astype(jnp.int32), 255)
                        + tile_base[g // 4] for g, px in zip(gs, pixs)]
                for idx in idxs:
                    plsc.addupdate_scatter(hist_v, [idx], ones)
            return 0
        lax.fori_loop(0, 64, row_body, 0)

        @pl.when(t + 2 < BPW)
        def _():
            pltpu.async_copy(in_slice(t + 2), buf, sem)

        @pl.when(t > 0)
        def _():
            pltpu.make_async_copy(
                lut_v, luts_hbm.at[pl.ds(band * LROW, LROW)], sem_o).wait()

        def tile_body(k, _):
            def merge_body(c, acc):
                base = k * NBINS + c * 16
                vs = [hist_v[pl.ds(l * LROW + base, 16)] for l in range(HCOPIES)]
                for l in range(HCOPIES):
                    hist_v[pl.ds(l * LROW + base, 16)] = zeros
                while len(vs) > 1:
                    vs = [a + b for a, b in zip(vs[0::2], vs[1::2])]
                s = jnp.minimum(vs[0], MAX_VAL)
                lut_v[pl.ds(base, 16)] = s
                return acc + s
            acc = lax.fori_loop(0, 16, merge_body, zeros)
            total = _take_last(plsc.cumsum(acc))
            clipped = PIXELS - total
            redist = (clipped * (1.0 / 256.0)).astype(jnp.int32).astype(jnp.float32)
            residual = clipped - redist * 256.0

            def lut_body(c, carry):
                base = k * NBINS + c * 16
                v = lut_v[pl.ds(base, 16)]
                binid = (c * 16 + lanes).astype(jnp.float32)
                v = v + redist + jnp.where(binid < residual, 1.0, 0.0)
                cumv = plsc.cumsum(v) + carry
                carry = _take_last(cumv)
                lut = (jnp.clip(cumv * LUT_SCALE, 0.0, 255.0)
                       ).astype(jnp.int32).astype(jnp.float32)
                lut_v[pl.ds(base, 16)] = lut
                return carry
            lax.fori_loop(0, 16, lut_body, zeros)
            return 0
        lax.fori_loop(0, 8, tile_body, 0)
        pltpu.async_copy(lut_v, luts_hbm.at[pl.ds(band * LROW, LROW)], sem_o)

    def pair_body(tt, _):
        half(2 * tt, band_a, sem_a)
        half(2 * tt + 1, band_b, sem_b)
        return 0
    lax.fori_loop(0, BPW // 2, pair_body, 0)
    pltpu.make_async_copy(lut_v, luts_hbm.at[pl.ds(0, LROW)], sem_o).wait()


@functools.partial(
    pl.kernel,
    out_type=jax.ShapeDtypeStruct((BANDS * BAND_ELEMS,), jnp.float32),
    mesh=_mesh,
    compiler_params=pltpu.CompilerParams(needs_layout_passes=False),
    scratch_types=[
        pltpu.VMEM((BAND_ELEMS,), jnp.float32),
        pltpu.VMEM((BAND_ELEMS,), jnp.float32),
        pltpu.VMEM((BAND_ELEMS,), jnp.float32),
        pltpu.VMEM((3 * LROW,), jnp.float32),
        pltpu.VMEM((3 * LROW,), jnp.float32),
        pltpu.VMEM((2 * LROW,), jnp.int32),
        pltpu.SemaphoreType.DMA,
        pltpu.SemaphoreType.DMA,
        pltpu.SemaphoreType.DMA,
    ],
)
def _apply_kernel(img_hbm, luts_hbm, out_hbm, band_a, band_b, out_v,
                  lr_a, lr_b, pck_v, sem_a, sem_b, sem_o):
    wid = lax.axis_index("s") * 2 + lax.axis_index("c")
    lanes = lax.iota(jnp.int32, 16)
    lanes_f = lanes.astype(jnp.float32)
    wxs = [(q * 16 + lanes_f + 0.5) * (1.0 / 64.0) + (0.5 if q < 2 else -0.5)
           for q in range(4)]
    himask = jnp.full((16,), -65536, jnp.int32)

    def in_copies(t, buf, lr, sem):
        band = wid * BPW + t
        p = band // 8
        j = band % 8
        jm = jnp.maximum(j - 1, 0)
        jp = jnp.minimum(j + 1, 7)
        return [
            (img_hbm.at[pl.ds(band * BAND_ELEMS, BAND_ELEMS)], buf, sem),
            (luts_hbm.at[pl.ds((p * 8 + jm) * LROW, LROW)],
             lr.at[pl.ds(0, LROW)], sem),
            (luts_hbm.at[pl.ds(band * LROW, LROW)],
             lr.at[pl.ds(LROW, LROW)], sem),
            (luts_hbm.at[pl.ds((p * 8 + jp) * LROW, LROW)],
             lr.at[pl.ds(2 * LROW, LROW)], sem),
        ]

    def start_in(t, buf, lr, sem):
        for src, dst, s in in_copies(t, buf, lr, sem):
            pltpu.async_copy(src, dst, s)

    def wait_in(t, buf, lr, sem):
        for src, dst, s in in_copies(t, buf, lr, sem):
            pltpu.make_async_copy(src, dst, s).wait()

    start_in(0, band_a, lr_a, sem_a)
    start_in(1, band_b, lr_b, sem_b)

    def half(t, buf, lr, sem):
        band = wid * BPW + t
        wait_in(t, buf, lr, sem)

        def pack_body(i, _):
            off = i * 32
            a0 = plsc.bitcast(lr[pl.ds(off, 16)], jnp.int32)
            a1 = plsc.bitcast(lr[pl.ds(off + 16, 16)], jnp.int32)
            b0 = plsc.bitcast(lr[pl.ds(LROW + off, 16)], jnp.int32)
            b1 = plsc.bitcast(lr[pl.ds(LROW + off + 16, 16)], jnp.int32)
            c0 = plsc.bitcast(lr[pl.ds(2 * LROW + off, 16)], jnp.int32)
            c1 = plsc.bitcast(lr[pl.ds(2 * LROW + off + 16, 16)], jnp.int32)
            pck_v[pl.ds(off, 16)] = (
                (a0 & himask) | jax.lax.shift_right_logical(b0, 16))
            pck_v[pl.ds(off + 16, 16)] = (
                (a1 & himask) | jax.lax.shift_right_logical(b1, 16))
            pck_v[pl.ds(LROW + off, 16)] = (
                (b0 & himask) | jax.lax.shift_right_logical(c0, 16))
            pck_v[pl.ds(LROW + off + 16, 16)] = (
                (b1 & himask) | jax.lax.shift_right_logical(c1, 16))
            return 0
        lax.fori_loop(0, LROW // 32, pack_body, 0)

        @pl.when(t > 0)
        def _():
            pltpu.make_async_copy(
                out_v, out_hbm.at[pl.ds(band * BAND_ELEMS, BAND_ELEMS)],
                sem_o).wait()

        def row_body(r, _):
            rbase = r * 512
            rv = jnp.full((16,), r, jnp.int32)
            ltv = rv < 32
            wy = (rv.astype(jnp.float32) * (1.0 / 64.0) + 0.0078125
                  + jnp.where(ltv, 0.5, -0.5))
            ybase = jnp.where(r < 32, 0, LROW)
            combo = [jnp.full((16,), ybase + x * NBINS, jnp.int32)
                     for x in range(8)]
            for g0 in range(0, 32, GROUP):
                gs = list(range(g0, g0 + GROUP))
                pixs = [buf[pl.ds(rbase + g * 16, 16)] for g in gs]
                bins = [jnp.minimum((px * 256.0).astype(jnp.int32), 255)
                        for px in pixs]
                w0s, w1s = [], []
                for g, b in zip(gs, bins):
                    k = g // 4
                    q = g % 4
                    x0 = k - 1 + q // 2
                    x0c = max(x0, 0)
                    x1c = min(x0 + 1, 7)
                    w0s.append(plsc.load_gather(pck_v, [b + combo[x0c]]))
                    w1s.append(plsc.load_gather(pck_v, [b + combo[x1c]]))
                for g, w0, w1 in zip(gs, w0s, w1s):
                    y00 = plsc.bitcast(w0 & himask, jnp.float32)
                    y10 = plsc.bitcast(jax.lax.shift_left(w0, 16), jnp.float32)
                    y01 = plsc.bitcast(w1 & himask, jnp.float32)
                    y11 = plsc.bitcast(jax.lax.shift_left(w1, 16), jnp.float32)
                    wx = wxs[g % 4]
                    top = y00 + wx * (y01 - y00)
                    bot = y10 + wx * (y11 - y10)
                    out_v[pl.ds(rbase + g * 16, 16)] = (
                        (top + wy * (bot - top)) * (1.0 / 255.0))
            return 0
        lax.fori_loop(0, 64, row_body, 0)
        pltpu.async_copy(
            out_v, out_hbm.at[pl.ds(band * BAND_ELEMS, BAND_ELEMS)], sem_o)

        @pl.when(t + 2 < BPW)
        def _():
            start_in(t + 2, buf, lr, sem)

    def pair_body(tt, _):
        half(2 * tt, band_a, lr_a, sem_a)
        half(2 * tt + 1, band_b, lr_b, sem_b)
        return 0
    lax.fori_loop(0, BPW // 2, pair_body, 0)
    pltpu.make_async_copy(
        out_v, out_hbm.at[pl.ds(0, BAND_ELEMS)], sem_o).wait()


def kernel(img):
    B, C, H, W = img.shape
    flat = img.reshape(BANDS * BAND_ELEMS)
    luts = _hist_lut_kernel(flat)
    out = _apply_kernel(flat, luts)
    return out.reshape(B, C, H, W)


# speedup vs baseline: 1920.5648x; 1.0474x over previous
import functools

import jax
import jax.numpy as jnp
from jax import lax
from jax.experimental import pallas as pl
from jax.experimental.pallas import tpu as pltpu
from jax.experimental.pallas import tpu_sc as plsc

NBINS = 256
MAX_VAL = 640.0
PIXELS = 4096.0
LUT_SCALE = 255.0 / 4096.0
NW = 32
BANDS = 384
BPW = BANDS // NW
BAND_ELEMS = 64 * 512
LROW = 8 * NBINS
HCOPIES = 16
GROUP = 16

_mesh = plsc.VectorSubcoreMesh(core_axis_name="c", subcore_axis_name="s")

_GATHER_DNUMS = lax.GatherDimensionNumbers(
    offset_dims=(), collapsed_slice_dims=(0,), start_index_map=(0,))


def _take_last(v):
    last = jnp.full((16, 1), 15, jnp.int32)
    return lax.gather(v, last, _GATHER_DNUMS, slice_sizes=(1,),
                      mode=lax.GatherScatterMode.PROMISE_IN_BOUNDS)


@functools.partial(
    pl.kernel,
    out_type=jax.ShapeDtypeStruct((BANDS * LROW,), jnp.float32),
    mesh=_mesh,
    compiler_params=pltpu.CompilerParams(needs_layout_passes=False),
    scratch_types=[
        pltpu.VMEM((BAND_ELEMS,), jnp.float32),
        pltpu.VMEM((BAND_ELEMS,), jnp.float32),
        pltpu.VMEM((HCOPIES * LROW,), jnp.float32),
        pltpu.VMEM((LROW,), jnp.float32),
        pltpu.SemaphoreType.DMA,
        pltpu.SemaphoreType.DMA,
        pltpu.SemaphoreType.DMA,
    ],
)
def _hist_lut_kernel(img_hbm, luts_hbm, band_a, band_b, hist_v, lut_v,
                     sem_a, sem_b, sem_o):
    wid = lax.axis_index("s") * 2 + lax.axis_index("c")
    lanes = lax.iota(jnp.int32, 16)
    lane_off = lanes * LROW
    tile_base = [lane_off + k * NBINS for k in range(8)]
    ones = jnp.ones((16,), jnp.float32)
    zeros = jnp.zeros((16,), jnp.float32)

    def in_slice(t):
        band = wid * BPW + t
        return img_hbm.at[pl.ds(band * BAND_ELEMS, BAND_ELEMS)]

    pltpu.async_copy(in_slice(0), band_a, sem_a)
    pltpu.async_copy(in_slice(1), band_b, sem_b)

    def zero_body(i, _):
        hist_v[pl.ds(i * 16, 16)] = zeros
        return 0
    lax.fori_loop(0, HCOPIES * LROW // 16, zero_body, 0)

    def half(t, buf, sem):
        band = wid * BPW + t
        pltpu.make_async_copy(in_slice(t), buf, sem).wait()

        def row_body(r, _):
            rbase = r * 512
            for g0 in range(0, 32, GROUP):
                gs = range(g0, g0 + GROUP)
                pixs = [buf[pl.ds(rbase + g * 16, 16)] for g in gs]
                idxs = [jnp.minimum((px * 256.0).astype(jnp.int32), 255)
                        + tile_base[g // 4] for g, px in zip(gs, pixs)]
                for idx in idxs:
                    plsc.addupdate_scatter(hist_v, [idx], ones)
            return 0
        lax.fori_loop(0, 64, row_body, 0)

        @pl.when(t + 2 < BPW)
        def _():
            pltpu.async_copy(in_slice(t + 2), buf, sem)

        @pl.when(t > 0)
        def _():
            pltpu.make_async_copy(
                lut_v, luts_hbm.at[pl.ds(band * LROW, LROW)], sem_o).wait()

        def tile_body(k, _):
            def merge_body(c, acc):
                base = k * NBINS + c * 16
                vs = [hist_v[pl.ds(l * LROW + base, 16)] for l in range(HCOPIES)]
                for l in range(HCOPIES):
                    hist_v[pl.ds(l * LROW + base, 16)] = zeros
                while len(vs) > 1:
                    vs = [a + b for a, b in zip(vs[0::2], vs[1::2])]
                s = jnp.minimum(vs[0], MAX_VAL)
                lut_v[pl.ds(base, 16)] = s
                return acc + s
            acc = lax.fori_loop(0, 16, merge_body, zeros)
            total = _take_last(plsc.cumsum(acc))
            clipped = PIXELS - total
            redist = (clipped * (1.0 / 256.0)).astype(jnp.int32).astype(jnp.float32)
            residual = clipped - redist * 256.0

            def lut_body(c, carry):
                base = k * NBINS + c * 16
                v = lut_v[pl.ds(base, 16)]
                binid = (c * 16 + lanes).astype(jnp.float32)
                v = v + redist + jnp.where(binid < residual, 1.0, 0.0)
                cumv = plsc.cumsum(v) + carry
                carry = _take_last(cumv)
                lut = (jnp.clip(cumv * LUT_SCALE, 0.0, 255.0)
                       ).astype(jnp.int32).astype(jnp.float32)
                lut_v[pl.ds(base, 16)] = lut
                return carry
            lax.fori_loop(0, 16, lut_body, zeros)
            return 0
        lax.fori_loop(0, 8, tile_body, 0)
        pltpu.async_copy(lut_v, luts_hbm.at[pl.ds(band * LROW, LROW)], sem_o)

    def pair_body(tt, _):
        half(2 * tt, band_a, sem_a)
        half(2 * tt + 1, band_b, sem_b)
        return 0
    lax.fori_loop(0, BPW // 2, pair_body, 0)
    pltpu.make_async_copy(lut_v, luts_hbm.at[pl.ds(0, LROW)], sem_o).wait()


@functools.partial(
    pl.kernel,
    out_type=jax.ShapeDtypeStruct((BANDS * BAND_ELEMS,), jnp.float32),
    mesh=_mesh,
    compiler_params=pltpu.CompilerParams(needs_layout_passes=False),
    scratch_types=[
        pltpu.VMEM((BAND_ELEMS,), jnp.float32),
        pltpu.VMEM((BAND_ELEMS,), jnp.float32),
        pltpu.VMEM((BAND_ELEMS,), jnp.float32),
        pltpu.VMEM((3 * LROW,), jnp.float32),
        pltpu.VMEM((3 * LROW,), jnp.float32),
        pltpu.VMEM((2 * LROW,), jnp.int32),
        pltpu.SemaphoreType.DMA,
        pltpu.SemaphoreType.DMA,
        pltpu.SemaphoreType.DMA,
    ],
)
def _apply_kernel(img_hbm, luts_hbm, out_hbm, band_a, band_b, out_v,
                  lr_a, lr_b, pck_v, sem_a, sem_b, sem_o):
    wid = lax.axis_index("s") * 2 + lax.axis_index("c")
    lanes = lax.iota(jnp.int32, 16)
    lanes_f = lanes.astype(jnp.float32)
    wxs = [(q * 16 + lanes_f + 0.5) * (1.0 / 64.0) + (0.5 if q < 2 else -0.5)
           for q in range(4)]
    himask = jnp.full((16,), -65536, jnp.int32)

    def in_copies(t, buf, lr, sem):
        band = wid * BPW + t
        p = band // 8
        j = band % 8
        jm = jnp.maximum(j - 1, 0)
        jp = jnp.minimum(j + 1, 7)
        return [
            (img_hbm.at[pl.ds(band * BAND_ELEMS, BAND_ELEMS)], buf, sem),
            (luts_hbm.at[pl.ds((p * 8 + jm) * LROW, LROW)],
             lr.at[pl.ds(0, LROW)], sem),
            (luts_hbm.at[pl.ds(band * LROW, LROW)],
             lr.at[pl.ds(LROW, LROW)], sem),
            (luts_hbm.at[pl.ds((p * 8 + jp) * LROW, LROW)],
             lr.at[pl.ds(2 * LROW, LROW)], sem),
        ]

    def start_in(t, buf, lr, sem):
        for src, dst, s in in_copies(t, buf, lr, sem):
            pltpu.async_copy(src, dst, s)

    def wait_in(t, buf, lr, sem):
        for src, dst, s in in_copies(t, buf, lr, sem):
            pltpu.make_async_copy(src, dst, s).wait()

    start_in(0, band_a, lr_a, sem_a)
    start_in(1, band_b, lr_b, sem_b)

    def half(t, buf, lr, sem):
        band = wid * BPW + t
        wait_in(t, buf, lr, sem)

        def pack_body(i, _):
            off = i * 32
            a0 = plsc.bitcast(lr[pl.ds(off, 16)], jnp.int32)
            a1 = plsc.bitcast(lr[pl.ds(off + 16, 16)], jnp.int32)
            b0 = plsc.bitcast(lr[pl.ds(LROW + off, 16)], jnp.int32)
            b1 = plsc.bitcast(lr[pl.ds(LROW + off + 16, 16)], jnp.int32)
            c0 = plsc.bitcast(lr[pl.ds(2 * LROW + off, 16)], jnp.int32)
            c1 = plsc.bitcast(lr[pl.ds(2 * LROW + off + 16, 16)], jnp.int32)
            pck_v[pl.ds(off, 16)] = (
                (a0 & himask) | jax.lax.shift_right_logical(b0, 16))
            pck_v[pl.ds(off + 16, 16)] = (
                (a1 & himask) | jax.lax.shift_right_logical(b1, 16))
            pck_v[pl.ds(LROW + off, 16)] = (
                (b0 & himask) | jax.lax.shift_right_logical(c0, 16))
            pck_v[pl.ds(LROW + off + 16, 16)] = (
                (b1 & himask) | jax.lax.shift_right_logical(c1, 16))
            return 0
        lax.fori_loop(0, LROW // 32, pack_body, 0)

        @pl.when(t > 0)
        def _():
            pltpu.make_async_copy(
                out_v, out_hbm.at[pl.ds(band * BAND_ELEMS, BAND_ELEMS)],
                sem_o).wait()

        def row_body(r, _):
            rbase = r * 512
            rv = jnp.full((16,), r, jnp.int32)
            ltv = rv < 32
            wy = (rv.astype(jnp.float32) * (1.0 / 64.0) + 0.0078125
                  + jnp.where(ltv, 0.5, -0.5))
            ybase = jnp.where(r < 32, 0, LROW)
            combo = [jnp.full((16,), ybase + x * NBINS, jnp.int32)
                     for x in range(8)]
            for g0 in range(0, 32, GROUP):
                gs = list(range(g0, g0 + GROUP))
                pixs = [buf[pl.ds(rbase + g * 16, 16)] for g in gs]
                bins = [jnp.minimum((px * 256.0).astype(jnp.int32), 255)
                        for px in pixs]
                w0s, w1s = [], []
                for g, b in zip(gs, bins):
                    k = g // 4
                    q = g % 4
                    x0 = k - 1 + q // 2
                    x0c = max(x0, 0)
                    x1c = min(x0 + 1, 7)
                    w0s.append(plsc.load_gather(pck_v, [b + combo[x0c]]))
                    w1s.append(plsc.load_gather(pck_v, [b + combo[x1c]]))
                for g, w0, w1 in zip(gs, w0s, w1s):
                    y00 = plsc.bitcast(w0 & himask, jnp.float32)
                    y10 = plsc.bitcast(jax.lax.shift_left(w0, 16), jnp.float32)
                    y01 = plsc.bitcast(w1 & himask, jnp.float32)
                    y11 = plsc.bitcast(jax.lax.shift_left(w1, 16), jnp.float32)
                    wx = wxs[g % 4]
                    top = y00 + wx * (y01 - y00)
                    bot = y10 + wx * (y11 - y10)
                    out_v[pl.ds(rbase + g * 16, 16)] = (
                        (top + wy * (bot - top)) * (1.0 / 255.0))
            return 0
        lax.fori_loop(0, 64, row_body, 0)
        pltpu.async_copy(
            out_v, out_hbm.at[pl.ds(band * BAND_ELEMS, BAND_ELEMS)], sem_o)

        @pl.when(t + 2 < BPW)
        def _():
            start_in(t + 2, buf, lr, sem)

    def pair_body(tt, _):
        half(2 * tt, band_a, lr_a, sem_a)
        half(2 * tt + 1, band_b, lr_b, sem_b)
        return 0
    lax.fori_loop(0, BPW // 2, pair_body, 0)
    pltpu.make_async_copy(
        out_v, out_hbm.at[pl.ds(0, BAND_ELEMS)], sem_o).wait()


def kernel(img):
    B, C, H, W = img.shape
    flat = img.reshape(BANDS * BAND_ELEMS)
    luts = _hist_lut_kernel(flat)
    out = _apply_kernel(flat, luts)
    return out.reshape(B, C, H, W)


# speedup vs baseline: 2533.6558x; 1.3192x over previous
import functools

import jax
import jax.numpy as jnp
from jax import lax
from jax.experimental import pallas as pl
from jax.experimental.pallas import tpu as pltpu
from jax.experimental.pallas import tpu_sc as plsc

NBINS = 256
MAX_VAL = 640.0
PIXELS = 4096.0
LUT_SCALE = 255.0 / 4096.0
NW = 32
BANDS = 384
BPW = BANDS // NW
BAND_ELEMS = 64 * 512
LROW = 8 * NBINS
HCOPIES = 16
GROUP = 16

_mesh = plsc.VectorSubcoreMesh(core_axis_name="c", subcore_axis_name="s")

_GATHER_DNUMS = lax.GatherDimensionNumbers(
    offset_dims=(), collapsed_slice_dims=(0,), start_index_map=(0,))


def _take_last(v):
    last = jnp.full((16, 1), 15, jnp.int32)
    return lax.gather(v, last, _GATHER_DNUMS, slice_sizes=(1,),
                      mode=lax.GatherScatterMode.PROMISE_IN_BOUNDS)


@functools.partial(
    pl.kernel,
    out_type=jax.ShapeDtypeStruct((BANDS * LROW,), jnp.float32),
    mesh=_mesh,
    compiler_params=pltpu.CompilerParams(needs_layout_passes=False),
    scratch_types=[
        pltpu.VMEM((64, 512), jnp.float32),
        pltpu.VMEM((64, 512), jnp.float32),
        pltpu.VMEM((HCOPIES * LROW,), jnp.float32),
        pltpu.VMEM((LROW,), jnp.float32),
        pltpu.SemaphoreType.DMA,
        pltpu.SemaphoreType.DMA,
        pltpu.SemaphoreType.DMA,
    ],
)
def _hist_lut_kernel(img_hbm, luts_hbm, band_a, band_b, hist_v, lut_v,
                     sem_a, sem_b, sem_o):
    wid = lax.axis_index("s") * 2 + lax.axis_index("c")
    lanes = lax.iota(jnp.int32, 16)
    lane_off = lanes * LROW
    tile_base = [lane_off + k * NBINS for k in range(8)]
    ones = jnp.ones((16,), jnp.float32)
    zeros = jnp.zeros((16,), jnp.float32)

    def in_slice(t):
        band = wid * BPW + t
        p = band // 8
        j = band % 8
        return img_hbm.at[p // 3, p % 3, pl.ds(j * 64, 64), :]

    pltpu.async_copy(in_slice(0), band_a, sem_a)
    pltpu.async_copy(in_slice(1), band_b, sem_b)

    def zero_body(i, _):
        hist_v[pl.ds(i * 16, 16)] = zeros
        return 0
    lax.fori_loop(0, HCOPIES * LROW // 16, zero_body, 0)

    def half(t, buf, sem):
        band = wid * BPW + t
        pltpu.make_async_copy(in_slice(t), buf, sem).wait()

        def row_body(r, _):
            for g0 in range(0, 32, GROUP):
                gs = range(g0, g0 + GROUP)
                pixs = [buf[r, pl.ds(g * 16, 16)] for g in gs]
                idxs = [jnp.minimum((px * 256.0).astype(jnp.int32), 255)
                        + tile_base[g // 4] for g, px in zip(gs, pixs)]
                for idx in idxs:
                    plsc.addupdate_scatter(hist_v, [idx], ones)
            return 0
        lax.fori_loop(0, 64, row_body, 0)

        @pl.when(t + 2 < BPW)
        def _():
            pltpu.async_copy(in_slice(t + 2), buf, sem)

        @pl.when(t > 0)
        def _():
            pltpu.make_async_copy(
                lut_v, luts_hbm.at[pl.ds(band * LROW, LROW)], sem_o).wait()

        def tile_body(k, _):
            def merge_body(c, acc):
                base = k * NBINS + c * 16
                vs = [hist_v[pl.ds(l * LROW + base, 16)] for l in range(HCOPIES)]
                for l in range(HCOPIES):
                    hist_v[pl.ds(l * LROW + base, 16)] = zeros
                while len(vs) > 1:
                    vs = [a + b for a, b in zip(vs[0::2], vs[1::2])]
                s = jnp.minimum(vs[0], MAX_VAL)
                lut_v[pl.ds(base, 16)] = s
                return acc + s
            acc = lax.fori_loop(0, 16, merge_body, zeros)
            total = _take_last(plsc.cumsum(acc))
            clipped = PIXELS - total
            redist = (clipped * (1.0 / 256.0)).astype(jnp.int32).astype(jnp.float32)
            residual = clipped - redist * 256.0

            def lut_body(c, carry):
                base = k * NBINS + c * 16
                v = lut_v[pl.ds(base, 16)]
                binid = (c * 16 + lanes).astype(jnp.float32)
                v = v + redist + jnp.where(binid < residual, 1.0, 0.0)
                cumv = plsc.cumsum(v) + carry
                carry = _take_last(cumv)
                lut = (jnp.clip(cumv * LUT_SCALE, 0.0, 255.0)
                       ).astype(jnp.int32).astype(jnp.float32)
                lut_v[pl.ds(base, 16)] = lut
                return carry
            lax.fori_loop(0, 16, lut_body, zeros)
            return 0
        lax.fori_loop(0, 8, tile_body, 0)
        pltpu.async_copy(lut_v, luts_hbm.at[pl.ds(band * LROW, LROW)], sem_o)

    def pair_body(tt, _):
        half(2 * tt, band_a, sem_a)
        half(2 * tt + 1, band_b, sem_b)
        return 0
    lax.fori_loop(0, BPW // 2, pair_body, 0)
    pltpu.make_async_copy(lut_v, luts_hbm.at[pl.ds(0, LROW)], sem_o).wait()


@functools.partial(
    pl.kernel,
    out_type=jax.ShapeDtypeStruct((16, 3, 512, 512), jnp.float32),
    mesh=_mesh,
    compiler_params=pltpu.CompilerParams(needs_layout_passes=False),
    scratch_types=[
        pltpu.VMEM((64, 512), jnp.float32),
        pltpu.VMEM((64, 512), jnp.float32),
        pltpu.VMEM((64, 512), jnp.float32),
        pltpu.VMEM((3 * LROW,), jnp.float32),
        pltpu.VMEM((3 * LROW,), jnp.float32),
        pltpu.VMEM((2 * LROW,), jnp.int32),
        pltpu.SemaphoreType.DMA,
        pltpu.SemaphoreType.DMA,
        pltpu.SemaphoreType.DMA,
    ],
)
def _apply_kernel(img_hbm, luts_hbm, out_hbm, band_a, band_b, out_v,
                  lr_a, lr_b, pck_v, sem_a, sem_b, sem_o):
    wid = lax.axis_index("s") * 2 + lax.axis_index("c")
    lanes = lax.iota(jnp.int32, 16)
    lanes_f = lanes.astype(jnp.float32)
    wxs = [(q * 16 + lanes_f + 0.5) * (1.0 / 64.0) + (0.5 if q < 2 else -0.5)
           for q in range(4)]
    himask = jnp.full((16,), -65536, jnp.int32)

    def in_copies(t, buf, lr, sem):
        band = wid * BPW + t
        p = band // 8
        j = band % 8
        jm = jnp.maximum(j - 1, 0)
        jp = jnp.minimum(j + 1, 7)
        return [
            (img_hbm.at[p // 3, p % 3, pl.ds(j * 64, 64), :], buf, sem),
            (luts_hbm.at[pl.ds((p * 8 + jm) * LROW, LROW)],
             lr.at[pl.ds(0, LROW)], sem),
            (luts_hbm.at[pl.ds(band * LROW, LROW)],
             lr.at[pl.ds(LROW, LROW)], sem),
            (luts_hbm.at[pl.ds((p * 8 + jp) * LROW, LROW)],
             lr.at[pl.ds(2 * LROW, LROW)], sem),
        ]

    def start_in(t, buf, lr, sem):
        for src, dst, s in in_copies(t, buf, lr, sem):
            pltpu.async_copy(src, dst, s)

    def wait_in(t, buf, lr, sem):
        for src, dst, s in in_copies(t, buf, lr, sem):
            pltpu.make_async_copy(src, dst, s).wait()

    start_in(0, band_a, lr_a, sem_a)
    start_in(1, band_b, lr_b, sem_b)

    def half(t, buf, lr, sem):
        band = wid * BPW + t
        wait_in(t, buf, lr, sem)

        def pack_body(i, _):
            off = i * 32
            a0 = plsc.bitcast(lr[pl.ds(off, 16)], jnp.int32)
            a1 = plsc.bitcast(lr[pl.ds(off + 16, 16)], jnp.int32)
            b0 = plsc.bitcast(lr[pl.ds(LROW + off, 16)], jnp.int32)
            b1 = plsc.bitcast(lr[pl.ds(LROW + off + 16, 16)], jnp.int32)
            c0 = plsc.bitcast(lr[pl.ds(2 * LROW + off, 16)], jnp.int32)
            c1 = plsc.bitcast(lr[pl.ds(2 * LROW + off + 16, 16)], jnp.int32)
            pck_v[pl.ds(off, 16)] = (
                (a0 & himask) | jax.lax.shift_right_logical(b0, 16))
            pck_v[pl.ds(off + 16, 16)] = (
                (a1 & himask) | jax.lax.shift_right_logical(b1, 16))
            pck_v[pl.ds(LROW + off, 16)] = (
                (b0 & himask) | jax.lax.shift_right_logical(c0, 16))
            pck_v[pl.ds(LROW + off + 16, 16)] = (
                (b1 & himask) | jax.lax.shift_right_logical(c1, 16))
            return 0
        lax.fori_loop(0, LROW // 32, pack_body, 0)

        p = band // 8
        j = band % 8

        @pl.when(t > 0)
        def _():
            pltpu.make_async_copy(
                out_v, out_hbm.at[p // 3, p % 3, pl.ds(j * 64, 64), :],
                sem_o).wait()

        def row_body(r, _):
            rv = jnp.full((16,), r, jnp.int32)
            ltv = rv < 32
            wy = (rv.astype(jnp.float32) * (1.0 / 64.0) + 0.0078125
                  + jnp.where(ltv, 0.5, -0.5))
            ybase = jnp.where(r < 32, 0, LROW)
            combo = [jnp.full((16,), ybase + x * NBINS, jnp.int32)
                     for x in range(8)]
            for g0 in range(0, 32, GROUP):
                gs = list(range(g0, g0 + GROUP))
                pixs = [buf[r, pl.ds(g * 16, 16)] for g in gs]
                bins = [jnp.minimum((px * 256.0).astype(jnp.int32), 255)
                        for px in pixs]
                w0s, w1s = [], []
                for g, b in zip(gs, bins):
                    k = g // 4
                    q = g % 4
                    x0 = k - 1 + q // 2
                    x0c = max(x0, 0)
                    x1c = min(x0 + 1, 7)
                    w0s.append(plsc.load_gather(pck_v, [b + combo[x0c]]))
                    w1s.append(plsc.load_gather(pck_v, [b + combo[x1c]]))
                for g, w0, w1 in zip(gs, w0s, w1s):
                    y00 = plsc.bitcast(w0 & himask, jnp.float32)
                    y10 = plsc.bitcast(jax.lax.shift_left(w0, 16), jnp.float32)
                    y01 = plsc.bitcast(w1 & himask, jnp.float32)
                    y11 = plsc.bitcast(jax.lax.shift_left(w1, 16), jnp.float32)
                    wx = wxs[g % 4]
                    top = y00 + wx * (y01 - y00)
                    bot = y10 + wx * (y11 - y10)
                    out_v[r, pl.ds(g * 16, 16)] = (
                        (top + wy * (bot - top)) * (1.0 / 255.0))
            return 0
        lax.fori_loop(0, 64, row_body, 0)
        pltpu.async_copy(
            out_v, out_hbm.at[p // 3, p % 3, pl.ds(j * 64, 64), :], sem_o)

        @pl.when(t + 2 < BPW)
        def _():
            start_in(t + 2, buf, lr, sem)

    def pair_body(tt, _):
        half(2 * tt, band_a, lr_a, sem_a)
        half(2 * tt + 1, band_b, lr_b, sem_b)
        return 0
    lax.fori_loop(0, BPW // 2, pair_body, 0)
    pltpu.make_async_copy(
        out_v, out_hbm.at[0, 0, pl.ds(0, 64), :], sem_o).wait()


def kernel(img):
    luts = _hist_lut_kernel(img)
    return _apply_kernel(img, luts)


# speedup vs baseline: 2803.1549x; 1.1064x over previous
import functools

import jax
import jax.numpy as jnp
from jax import lax
from jax.experimental import pallas as pl
from jax.experimental.pallas import tpu as pltpu
from jax.experimental.pallas import tpu_sc as plsc

NBINS = 256
MAX_VAL = 640.0
PIXELS = 4096.0
LUT_SCALE = 255.0 / 4096.0
NW = 32
BANDS = 384
BPW = BANDS // NW
BAND_ELEMS = 64 * 512
LROW = 8 * NBINS
HCOPIES = 16
GROUP = 16

_mesh = plsc.VectorSubcoreMesh(core_axis_name="c", subcore_axis_name="s")

_GATHER_DNUMS = lax.GatherDimensionNumbers(
    offset_dims=(), collapsed_slice_dims=(0,), start_index_map=(0,))


def _take_last(v):
    last = jnp.full((16, 1), 15, jnp.int32)
    return lax.gather(v, last, _GATHER_DNUMS, slice_sizes=(1,),
                      mode=lax.GatherScatterMode.PROMISE_IN_BOUNDS)


@functools.partial(
    pl.kernel,
    out_type=jax.ShapeDtypeStruct((BANDS * LROW,), jnp.float32),
    mesh=_mesh,
    compiler_params=pltpu.CompilerParams(needs_layout_passes=False),
    scratch_types=[
        pltpu.VMEM((64, 512), jnp.float32),
        pltpu.VMEM((64, 512), jnp.float32),
        pltpu.VMEM((LROW,), jnp.float32),
        pltpu.VMEM((LROW,), jnp.float32),
        pltpu.SemaphoreType.DMA,
        pltpu.SemaphoreType.DMA,
        pltpu.SemaphoreType.DMA,
    ],
)
def _hist_lut_kernel(img_hbm, luts_hbm, band_a, band_b, hist_v, lut_v,
                     sem_a, sem_b, sem_o):
    wid = lax.axis_index("s") * 2 + lax.axis_index("c")
    lanes = lax.iota(jnp.int32, 16)
    tile_base = [jnp.full((16,), k * NBINS, jnp.int32) for k in range(8)]
    ones = jnp.ones((16,), jnp.float32)
    zeros = jnp.zeros((16,), jnp.float32)

    def in_slice(t):
        band = wid * BPW + t
        p = band // 8
        j = band % 8
        return img_hbm.at[p // 3, p % 3, pl.ds(j * 64, 64), :]

    pltpu.async_copy(in_slice(0), band_a, sem_a)
    pltpu.async_copy(in_slice(1), band_b, sem_b)

    def zero_body(i, _):
        hist_v[pl.ds(i * 16, 16)] = zeros
        return 0
    lax.fori_loop(0, LROW // 16, zero_body, 0)

    def half(t, buf, sem):
        band = wid * BPW + t
        pltpu.make_async_copy(in_slice(t), buf, sem).wait()

        def row_body(r, _):
            for g0 in range(0, 32, GROUP):
                gs = range(g0, g0 + GROUP)
                pixs = [buf[r, pl.ds(g * 16, 16)] for g in gs]
                idxs = [jnp.minimum((px * 256.0).astype(jnp.int32), 255)
                        + tile_base[g // 4] for g, px in zip(gs, pixs)]
                for idx in idxs:
                    plsc.addupdate_scatter(hist_v, [idx], ones)
            return 0
        lax.fori_loop(0, 64, row_body, 0)

        @pl.when(t + 2 < BPW)
        def _():
            pltpu.async_copy(in_slice(t + 2), buf, sem)

        @pl.when(t > 0)
        def _():
            pltpu.make_async_copy(
                lut_v, luts_hbm.at[pl.ds(band * LROW, LROW)], sem_o).wait()

        def tile_body(k, _):
            def merge_body(c, acc):
                base = k * NBINS + c * 16
                s = hist_v[pl.ds(base, 16)]
                hist_v[pl.ds(base, 16)] = zeros
                s = jnp.minimum(s, MAX_VAL)
                lut_v[pl.ds(base, 16)] = s
                return acc + s
            acc = lax.fori_loop(0, 16, merge_body, zeros)
            total = _take_last(plsc.cumsum(acc))
            clipped = PIXELS - total
            redist = (clipped * (1.0 / 256.0)).astype(jnp.int32).astype(jnp.float32)
            residual = clipped - redist * 256.0

            def lut_body(c, carry):
                base = k * NBINS + c * 16
                v = lut_v[pl.ds(base, 16)]
                binid = (c * 16 + lanes).astype(jnp.float32)
                v = v + redist + jnp.where(binid < residual, 1.0, 0.0)
                cumv = plsc.cumsum(v) + carry
                carry = _take_last(cumv)
                lut = (jnp.clip(cumv * LUT_SCALE, 0.0, 255.0)
                       ).astype(jnp.int32).astype(jnp.float32)
                lut_v[pl.ds(base, 16)] = lut
                return carry
            lax.fori_loop(0, 16, lut_body, zeros)
            return 0
        lax.fori_loop(0, 8, tile_body, 0)
        pltpu.async_copy(lut_v, luts_hbm.at[pl.ds(band * LROW, LROW)], sem_o)

    def pair_body(tt, _):
        half(2 * tt, band_a, sem_a)
        half(2 * tt + 1, band_b, sem_b)
        return 0
    lax.fori_loop(0, BPW // 2, pair_body, 0)
    pltpu.make_async_copy(lut_v, luts_hbm.at[pl.ds(0, LROW)], sem_o).wait()


@functools.partial(
    pl.kernel,
    out_type=jax.ShapeDtypeStruct((16, 3, 512, 512), jnp.float32),
    mesh=_mesh,
    compiler_params=pltpu.CompilerParams(needs_layout_passes=False),
    scratch_types=[
        pltpu.VMEM((64, 512), jnp.float32),
        pltpu.VMEM((64, 512), jnp.float32),
        pltpu.VMEM((64, 512), jnp.float32),
        pltpu.VMEM((3 * LROW,), jnp.float32),
        pltpu.VMEM((3 * LROW,), jnp.float32),
        pltpu.VMEM((2 * LROW,), jnp.int32),
        pltpu.SemaphoreType.DMA,
        pltpu.SemaphoreType.DMA,
        pltpu.SemaphoreType.DMA,
    ],
)
def _apply_kernel(img_hbm, luts_hbm, out_hbm, band_a, band_b, out_v,
                  lr_a, lr_b, pck_v, sem_a, sem_b, sem_o):
    wid = lax.axis_index("s") * 2 + lax.axis_index("c")
    lanes = lax.iota(jnp.int32, 16)
    lanes_f = lanes.astype(jnp.float32)
    wxs = [(q * 16 + lanes_f + 0.5) * (1.0 / 64.0) + (0.5 if q < 2 else -0.5)
           for q in range(4)]
    himask = jnp.full((16,), -65536, jnp.int32)

    def in_copies(t, buf, lr, sem):
        band = wid * BPW + t
        p = band // 8
        j = band % 8
        jm = jnp.maximum(j - 1, 0)
        jp = jnp.minimum(j + 1, 7)
        return [
            (img_hbm.at[p // 3, p % 3, pl.ds(j * 64, 64), :], buf, sem),
            (luts_hbm.at[pl.ds((p * 8 + jm) * LROW, LROW)],
             lr.at[pl.ds(0, LROW)], sem),
            (luts_hbm.at[pl.ds(band * LROW, LROW)],
             lr.at[pl.ds(LROW, LROW)], sem),
            (luts_hbm.at[pl.ds((p * 8 + jp) * LROW, LROW)],
             lr.at[pl.ds(2 * LROW, LROW)], sem),
        ]

    def start_in(t, buf, lr, sem):
        for src, dst, s in in_copies(t, buf, lr, sem):
            pltpu.async_copy(src, dst, s)

    def wait_in(t, buf, lr, sem):
        for src, dst, s in in_copies(t, buf, lr, sem):
            pltpu.make_async_copy(src, dst, s).wait()

    start_in(0, band_a, lr_a, sem_a)
    start_in(1, band_b, lr_b, sem_b)

    def half(t, buf, lr, sem):
        band = wid * BPW + t
        wait_in(t, buf, lr, sem)

        def pack_body(i, _):
            off = i * 32
            a0 = plsc.bitcast(lr[pl.ds(off, 16)], jnp.int32)
            a1 = plsc.bitcast(lr[pl.ds(off + 16, 16)], jnp.int32)
            b0 = plsc.bitcast(lr[pl.ds(LROW + off, 16)], jnp.int32)
            b1 = plsc.bitcast(lr[pl.ds(LROW + off + 16, 16)], jnp.int32)
            c0 = plsc.bitcast(lr[pl.ds(2 * LROW + off, 16)], jnp.int32)
            c1 = plsc.bitcast(lr[pl.ds(2 * LROW + off + 16, 16)], jnp.int32)
            pck_v[pl.ds(off, 16)] = (
                (a0 & himask) | jax.lax.shift_right_logical(b0, 16))
            pck_v[pl.ds(off + 16, 16)] = (
                (a1 & himask) | jax.lax.shift_right_logical(b1, 16))
            pck_v[pl.ds(LROW + off, 16)] = (
                (b0 & himask) | jax.lax.shift_right_logical(c0, 16))
            pck_v[pl.ds(LROW + off + 16, 16)] = (
                (b1 & himask) | jax.lax.shift_right_logical(c1, 16))
            return 0
        lax.fori_loop(0, LROW // 32, pack_body, 0)

        p = band // 8
        j = band % 8

        @pl.when(t > 0)
        def _():
            pltpu.make_async_copy(
                out_v, out_hbm.at[p // 3, p % 3, pl.ds(j * 64, 64), :],
                sem_o).wait()

        def row_body(r, _):
            rv = jnp.full((16,), r, jnp.int32)
            ltv = rv < 32
            wy = (rv.astype(jnp.float32) * (1.0 / 64.0) + 0.0078125
                  + jnp.where(ltv, 0.5, -0.5))
            ybase = jnp.where(r < 32, 0, LROW)
            combo = [jnp.full((16,), ybase + x * NBINS, jnp.int32)
                     for x in range(8)]
            for g0 in range(0, 32, GROUP):
                gs = list(range(g0, g0 + GROUP))
                pixs = [buf[r, pl.ds(g * 16, 16)] for g in gs]
                bins = [jnp.minimum((px * 256.0).astype(jnp.int32), 255)
                        for px in pixs]
                w0s, w1s = [], []
                for g, b in zip(gs, bins):
                    k = g // 4
                    q = g % 4
                    x0 = k - 1 + q // 2
                    x0c = max(x0, 0)
                    x1c = min(x0 + 1, 7)
                    w0s.append(plsc.load_gather(pck_v, [b + combo[x0c]]))
                    w1s.append(plsc.load_gather(pck_v, [b + combo[x1c]]))
                for g, w0, w1 in zip(gs, w0s, w1s):
                    y00 = plsc.bitcast(w0 & himask, jnp.float32)
                    y10 = plsc.bitcast(jax.lax.shift_left(w0, 16), jnp.float32)
                    y01 = plsc.bitcast(w1 & himask, jnp.float32)
                    y11 = plsc.bitcast(jax.lax.shift_left(w1, 16), jnp.float32)
                    wx = wxs[g % 4]
                    top = y00 + wx * (y01 - y00)
                    bot = y10 + wx * (y11 - y10)
                    out_v[r, pl.ds(g * 16, 16)] = (
                        (top + wy * (bot - top)) * (1.0 / 255.0))
            return 0
        lax.fori_loop(0, 64, row_body, 0)
        pltpu.async_copy(
            out_v, out_hbm.at[p // 3, p % 3, pl.ds(j * 64, 64), :], sem_o)

        @pl.when(t + 2 < BPW)
        def _():
            start_in(t + 2, buf, lr, sem)

    def pair_body(tt, _):
        half(2 * tt, band_a, lr_a, sem_a)
        half(2 * tt + 1, band_b, lr_b, sem_b)
        return 0
    lax.fori_loop(0, BPW // 2, pair_body, 0)
    pltpu.make_async_copy(
        out_v, out_hbm.at[0, 0, pl.ds(0, 64), :], sem_o).wait()


def kernel(img):
    luts = _hist_lut_kernel(img)
    return _apply_kernel(img, luts)
